# speedup vs baseline: 1.1817x; 1.1817x over previous
import functools

import jax
import jax.numpy as jnp
from jax import lax
from jax.experimental import pallas as pl

_EPS = 1e-5


def _dotT(a, w):
    return lax.dot_general(a, w, (((1,), (1,)), ((), ())),
                           preferred_element_type=jnp.float32)


def _precompute_body(x_ref, wsg_ref, wdg_ref, wsm_ref, bsm_ref,
                     gs_ref, gd_ref, ms_ref):
    xb = x_ref[...]
    gs_ref[...] = _dotT(xb, wsg_ref[...])
    gd_ref[...] = _dotT(xb, wdg_ref[...])
    ms_ref[...] = _dotT(xb, wsm_ref[...]) + bsm_ref[...]


def _precompute(x, wsg, wdg, wsm, bsm, blk):
    n, din = x.shape
    de = wsg.shape[0]
    dn = wsm.shape[0]
    grid = (n // blk,)
    return pl.pallas_call(
        _precompute_body,
        grid=grid,
        in_specs=[
            pl.BlockSpec((blk, din), lambda i: (i, 0)),
            pl.BlockSpec((de, din), lambda i: (0, 0)),
            pl.BlockSpec((de, din), lambda i: (0, 0)),
            pl.BlockSpec((dn, din), lambda i: (0, 0)),
            pl.BlockSpec((1, dn), lambda i: (0, 0)),
        ],
        out_specs=[
            pl.BlockSpec((blk, de), lambda i: (i, 0)),
            pl.BlockSpec((blk, de), lambda i: (i, 0)),
            pl.BlockSpec((blk, dn), lambda i: (i, 0)),
        ],
        out_shape=[
            jax.ShapeDtypeStruct((n, de), jnp.float32),
            jax.ShapeDtypeStruct((n, de), jnp.float32),
            jax.ShapeDtypeStruct((n, dn), jnp.float32),
        ],
    )(x, wsg, wdg, wsm, bsm.reshape(1, -1))


def _gate_body(ea_ref, gsr_ref, gdc_ref, weg_ref, bg_ref,
               h_ref, sum_ref, ssq_ref):
    i = pl.program_id(0)
    ea = ea_ref[...]
    z = gsr_ref[...] + gdc_ref[...] + _dotT(ea, weg_ref[...]) + bg_ref[...]
    u = jax.nn.sigmoid(z) * ea
    h = u * jax.nn.sigmoid(u)
    h_ref[...] = h

    @pl.when(i == 0)
    def _():
        sum_ref[...] = jnp.zeros_like(sum_ref)
        ssq_ref[...] = jnp.zeros_like(ssq_ref)

    sum_ref[...] += jnp.sum(h, axis=0, keepdims=True)
    ssq_ref[...] += jnp.sum(h * h, axis=0, keepdims=True)


def _gate_pass(ea, gsr, gdc, weg, bg, blk):
    ne, de = ea.shape
    grid = (ne // blk,)
    return pl.pallas_call(
        _gate_body,
        grid=grid,
        in_specs=[
            pl.BlockSpec((blk, de), lambda i: (i, 0)),
            pl.BlockSpec((blk, de), lambda i: (i, 0)),
            pl.BlockSpec((blk, de), lambda i: (i, 0)),
            pl.BlockSpec((de, de), lambda i: (0, 0)),
            pl.BlockSpec((1, de), lambda i: (0, 0)),
        ],
        out_specs=[
            pl.BlockSpec((blk, de), lambda i: (i, 0)),
            pl.BlockSpec((1, de), lambda i: (0, 0)),
            pl.BlockSpec((1, de), lambda i: (0, 0)),
        ],
        out_shape=[
            jax.ShapeDtypeStruct((ne, de), jnp.float32),
            jax.ShapeDtypeStruct((1, de), jnp.float32),
            jax.ShapeDtypeStruct((1, de), jnp.float32),
        ],
    )(ea, gsr, gdc, weg, bg.reshape(1, -1))


def _msg_body(h_ref, ea_ref, scale_ref, shift_ref, wem_ref, bem_ref,
              eout_ref, msge_ref):
    en = h_ref[...] * scale_ref[...] + shift_ref[...]
    eout_ref[...] = ea_ref[...] + en
    msge_ref[...] = _dotT(en, wem_ref[...]) + bem_ref[...]


def _msg_pass(h, ea, scale, shift, wem, bem, blk):
    ne, de = h.shape
    dn = wem.shape[0]
    grid = (ne // blk,)
    return pl.pallas_call(
        _msg_body,
        grid=grid,
        in_specs=[
            pl.BlockSpec((blk, de), lambda i: (i, 0)),
            pl.BlockSpec((blk, de), lambda i: (i, 0)),
            pl.BlockSpec((1, de), lambda i: (0, 0)),
            pl.BlockSpec((1, de), lambda i: (0, 0)),
            pl.BlockSpec((dn, de), lambda i: (0, 0)),
            pl.BlockSpec((1, dn), lambda i: (0, 0)),
        ],
        out_specs=[
            pl.BlockSpec((blk, de), lambda i: (i, 0)),
            pl.BlockSpec((blk, dn), lambda i: (i, 0)),
        ],
        out_shape=[
            jax.ShapeDtypeStruct((ne, de), jnp.float32),
            jax.ShapeDtypeStruct((ne, dn), jnp.float32),
        ],
    )(h, ea, scale.reshape(1, -1), shift.reshape(1, -1), wem,
      bem.reshape(1, -1))


def _silu_stats_body(agg_ref, s_ref, sum_ref, ssq_ref):
    i = pl.program_id(0)
    a = agg_ref[...]
    s = a * jax.nn.sigmoid(a)
    s_ref[...] = s

    @pl.when(i == 0)
    def _():
        sum_ref[...] = jnp.zeros_like(sum_ref)
        ssq_ref[...] = jnp.zeros_like(ssq_ref)

    sum_ref[...] += jnp.sum(s, axis=0, keepdims=True)
    ssq_ref[...] += jnp.sum(s * s, axis=0, keepdims=True)


def _silu_stats(agg, blk):
    n, d = agg.shape
    grid = (n // blk,)
    return pl.pallas_call(
        _silu_stats_body,
        grid=grid,
        in_specs=[pl.BlockSpec((blk, d), lambda i: (i, 0))],
        out_specs=[
            pl.BlockSpec((blk, d), lambda i: (i, 0)),
            pl.BlockSpec((1, d), lambda i: (0, 0)),
            pl.BlockSpec((1, d), lambda i: (0, 0)),
        ],
        out_shape=[
            jax.ShapeDtypeStruct((n, d), jnp.float32),
            jax.ShapeDtypeStruct((1, d), jnp.float32),
            jax.ShapeDtypeStruct((1, d), jnp.float32),
        ],
    )(agg)


def _norm_res_body(s_ref, x_ref, scale_ref, shift_ref, out_ref):
    out_ref[...] = x_ref[...] + s_ref[...] * scale_ref[...] + shift_ref[...]


def _norm_res(s, x, scale, shift, blk):
    n, d = s.shape
    grid = (n // blk,)
    return pl.pallas_call(
        _norm_res_body,
        grid=grid,
        in_specs=[
            pl.BlockSpec((blk, d), lambda i: (i, 0)),
            pl.BlockSpec((blk, d), lambda i: (i, 0)),
            pl.BlockSpec((1, d), lambda i: (0, 0)),
            pl.BlockSpec((1, d), lambda i: (0, 0)),
        ],
        out_specs=pl.BlockSpec((blk, d), lambda i: (i, 0)),
        out_shape=jax.ShapeDtypeStruct((n, d), jnp.float32),
    )(s, x, scale.reshape(1, -1), shift.reshape(1, -1))


def _bn_scale_shift(sumv, ssqv, count, g, b):
    mean = (sumv / count).reshape(-1)
    var = (ssqv / count).reshape(-1) - mean * mean
    inv = lax.rsqrt(var + _EPS)
    scale = g * inv
    shift = b - mean * scale
    return scale, shift


def _edge_gated_conv(p, x, edge_index, edge_attr, num_nodes,
                     blk_node, blk_edge):
    row = edge_index[0].astype(jnp.int32)
    col = edge_index[1].astype(jnp.int32)
    ne = edge_attr.shape[0]

    bg = p["src_gate"]["b"] + p["dst_gate"]["b"] + p["edge_gate"]["b"]
    gs, gd, ms = _precompute(x, p["src_gate"]["W"], p["dst_gate"]["W"],
                             p["src_msg"]["W"], p["src_msg"]["b"], blk_node)

    gsr = jnp.take(gs, row, axis=0)
    gdc = jnp.take(gd, col, axis=0)

    h, hsum, hssq = _gate_pass(edge_attr, gsr, gdc, p["edge_gate"]["W"], bg,
                               blk_edge)
    e_scale, e_shift = _bn_scale_shift(hsum, hssq, float(ne),
                                       p["bn_edge"]["g"], p["bn_edge"]["b"])

    e_out, msg_e = _msg_pass(h, edge_attr, e_scale, e_shift,
                             p["edge_msg"]["W"], p["edge_msg"]["b"], blk_edge)

    msg = jnp.take(ms, row, axis=0) + msg_e
    agg = jax.ops.segment_sum(msg, col, num_segments=num_nodes)

    s, ssum, sssq = _silu_stats(agg, blk_node)
    n_scale, n_shift = _bn_scale_shift(ssum, sssq, float(num_nodes),
                                       p["bn_node"]["g"], p["bn_node"]["b"])
    x_out = _norm_res(s, x, n_scale, n_shift, blk_node)
    return x_out, e_out


def kernel(x, edge_index, edge_attr, line_index, line_attr, params):
    ea, la = _edge_gated_conv(params["line_conv"], edge_attr, line_index,
                              line_attr, edge_attr.shape[0],
                              blk_node=640, blk_edge=640)
    xo, ea = _edge_gated_conv(params["atom_conv"], x, edge_index, ea,
                              x.shape[0], blk_node=1000, blk_edge=640)
    return (xo, ea, la)


# speedup vs baseline: 1.2530x; 1.0604x over previous
import functools

import jax
import jax.numpy as jnp
from jax import lax
from jax.experimental import pallas as pl
from jax.experimental.pallas import tpu as pltpu
from jax.experimental.pallas import tpu_sc as plsc

_EPS = 1e-5

_NC = 2
_NS = 16
_NW = _NC * _NS


def _dotT(a, w):
    return lax.dot_general(a, w, (((1,), (1,)), ((), ())),
                           preferred_element_type=jnp.float32)


def _precompute_body(x_ref, wsg_ref, wdg_ref, wsm_ref, bsm_ref,
                     gs_ref, gd_ref, ms_ref):
    xb = x_ref[...]
    gs_ref[...] = _dotT(xb, wsg_ref[...])
    gd_ref[...] = _dotT(xb, wdg_ref[...])
    ms_ref[...] = _dotT(xb, wsm_ref[...]) + bsm_ref[...]


def _precompute(x, wsg, wdg, wsm, bsm, blk):
    n, din = x.shape
    de = wsg.shape[0]
    dn = wsm.shape[0]
    grid = (n // blk,)
    return pl.pallas_call(
        _precompute_body,
        grid=grid,
        in_specs=[
            pl.BlockSpec((blk, din), lambda i: (i, 0)),
            pl.BlockSpec((de, din), lambda i: (0, 0)),
            pl.BlockSpec((de, din), lambda i: (0, 0)),
            pl.BlockSpec((dn, din), lambda i: (0, 0)),
            pl.BlockSpec((1, dn), lambda i: (0, 0)),
        ],
        out_specs=[
            pl.BlockSpec((blk, de), lambda i: (i, 0)),
            pl.BlockSpec((blk, de), lambda i: (i, 0)),
            pl.BlockSpec((blk, dn), lambda i: (i, 0)),
        ],
        out_shape=[
            jax.ShapeDtypeStruct((n, de), jnp.float32),
            jax.ShapeDtypeStruct((n, de), jnp.float32),
            jax.ShapeDtypeStruct((n, dn), jnp.float32),
        ],
    )(x, wsg, wdg, wsm, bsm.reshape(1, -1))


def _gate_body(ea_ref, gsr_ref, gdc_ref, weg_ref, bg_ref,
               h_ref, sum_ref, ssq_ref):
    i = pl.program_id(0)
    ea = ea_ref[...]
    z = gsr_ref[...] + gdc_ref[...] + _dotT(ea, weg_ref[...]) + bg_ref[...]
    u = jax.nn.sigmoid(z) * ea
    h = u * jax.nn.sigmoid(u)
    h_ref[...] = h

    @pl.when(i == 0)
    def _():
        sum_ref[...] = jnp.zeros_like(sum_ref)
        ssq_ref[...] = jnp.zeros_like(ssq_ref)

    sum_ref[...] += jnp.sum(h, axis=0, keepdims=True)
    ssq_ref[...] += jnp.sum(h * h, axis=0, keepdims=True)


def _gate_pass(ea, gsr, gdc, weg, bg, blk):
    ne, de = ea.shape
    grid = (ne // blk,)
    return pl.pallas_call(
        _gate_body,
        grid=grid,
        in_specs=[
            pl.BlockSpec((blk, de), lambda i: (i, 0)),
            pl.BlockSpec((blk, de), lambda i: (i, 0)),
            pl.BlockSpec((blk, de), lambda i: (i, 0)),
            pl.BlockSpec((de, de), lambda i: (0, 0)),
            pl.BlockSpec((1, de), lambda i: (0, 0)),
        ],
        out_specs=[
            pl.BlockSpec((blk, de), lambda i: (i, 0)),
            pl.BlockSpec((1, de), lambda i: (0, 0)),
            pl.BlockSpec((1, de), lambda i: (0, 0)),
        ],
        out_shape=[
            jax.ShapeDtypeStruct((ne, de), jnp.float32),
            jax.ShapeDtypeStruct((1, de), jnp.float32),
            jax.ShapeDtypeStruct((1, de), jnp.float32),
        ],
    )(ea, gsr, gdc, weg, bg.reshape(1, -1))


def _msg_body(h_ref, ea_ref, scale_ref, shift_ref, wem_ref, bem_ref,
              eout_ref, msge_ref):
    en = h_ref[...] * scale_ref[...] + shift_ref[...]
    eout_ref[...] = ea_ref[...] + en
    msge_ref[...] = _dotT(en, wem_ref[...]) + bem_ref[...]


def _msg_pass(h, ea, scale, shift, wem, bem, blk):
    ne, de = h.shape
    dn = wem.shape[0]
    grid = (ne // blk,)
    return pl.pallas_call(
        _msg_body,
        grid=grid,
        in_specs=[
            pl.BlockSpec((blk, de), lambda i: (i, 0)),
            pl.BlockSpec((blk, de), lambda i: (i, 0)),
            pl.BlockSpec((1, de), lambda i: (0, 0)),
            pl.BlockSpec((1, de), lambda i: (0, 0)),
            pl.BlockSpec((dn, de), lambda i: (0, 0)),
            pl.BlockSpec((1, dn), lambda i: (0, 0)),
        ],
        out_specs=[
            pl.BlockSpec((blk, de), lambda i: (i, 0)),
            pl.BlockSpec((blk, dn), lambda i: (i, 0)),
        ],
        out_shape=[
            jax.ShapeDtypeStruct((ne, de), jnp.float32),
            jax.ShapeDtypeStruct((ne, dn), jnp.float32),
        ],
    )(h, ea, scale.reshape(1, -1), shift.reshape(1, -1), wem,
      bem.reshape(1, -1))


def _silu_stats_body(agg_ref, s_ref, sum_ref, ssq_ref):
    i = pl.program_id(0)
    a = agg_ref[...]
    s = a * jax.nn.sigmoid(a)
    s_ref[...] = s

    @pl.when(i == 0)
    def _():
        sum_ref[...] = jnp.zeros_like(sum_ref)
        ssq_ref[...] = jnp.zeros_like(ssq_ref)

    sum_ref[...] += jnp.sum(s, axis=0, keepdims=True)
    ssq_ref[...] += jnp.sum(s * s, axis=0, keepdims=True)


def _silu_stats(agg, blk):
    n, d = agg.shape
    grid = (n // blk,)
    return pl.pallas_call(
        _silu_stats_body,
        grid=grid,
        in_specs=[pl.BlockSpec((blk, d), lambda i: (i, 0))],
        out_specs=[
            pl.BlockSpec((blk, d), lambda i: (i, 0)),
            pl.BlockSpec((1, d), lambda i: (0, 0)),
            pl.BlockSpec((1, d), lambda i: (0, 0)),
        ],
        out_shape=[
            jax.ShapeDtypeStruct((n, d), jnp.float32),
            jax.ShapeDtypeStruct((1, d), jnp.float32),
            jax.ShapeDtypeStruct((1, d), jnp.float32),
        ],
    )(agg)


def _norm_res_body(s_ref, x_ref, scale_ref, shift_ref, out_ref):
    out_ref[...] = x_ref[...] + s_ref[...] * scale_ref[...] + shift_ref[...]


def _norm_res(s, x, scale, shift, blk):
    n, d = s.shape
    grid = (n // blk,)
    return pl.pallas_call(
        _norm_res_body,
        grid=grid,
        in_specs=[
            pl.BlockSpec((blk, d), lambda i: (i, 0)),
            pl.BlockSpec((blk, d), lambda i: (i, 0)),
            pl.BlockSpec((1, d), lambda i: (0, 0)),
            pl.BlockSpec((1, d), lambda i: (0, 0)),
        ],
        out_specs=pl.BlockSpec((blk, d), lambda i: (i, 0)),
        out_shape=jax.ShapeDtypeStruct((n, d), jnp.float32),
    )(s, x, scale.reshape(1, -1), shift.reshape(1, -1))


def _sc_gather_pair(ta, tb, ia, ib, sub):
    b_total = ia.shape[0]
    da = ta.shape[1]
    db = tb.shape[1]
    b_per_w = b_total // _NW
    n_sub = b_per_w // sub
    mesh = plsc.VectorSubcoreMesh(core_axis_name="c", subcore_axis_name="s")

    @functools.partial(
        pl.kernel,
        out_type=[
            jax.ShapeDtypeStruct((b_total, da), jnp.float32),
            jax.ShapeDtypeStruct((b_total, db), jnp.float32),
        ],
        mesh=mesh,
        scratch_types=[
            pltpu.VMEM((1, sub), jnp.int32),
            pltpu.VMEM((1, sub), jnp.int32),
            pltpu.VMEM((sub, da), jnp.float32),
            pltpu.VMEM((sub, db), jnp.float32),
            pltpu.SemaphoreType.DMA,
            pltpu.SemaphoreType.DMA,
        ],
    )
    def gather_k(ta_hbm, tb_hbm, ia_hbm, ib_hbm, oa_hbm, ob_hbm,
                 ia_v, ib_v, ra_v, rb_v, sema, semb):
        wid = lax.axis_index("s") * _NC + lax.axis_index("c")
        base = wid * b_per_w

        def body(j, carry):
            off = pl.multiple_of(base + j * sub, 8)
            pltpu.sync_copy(ia_hbm.at[pl.ds(off, sub)], ia_v.at[0])
            pltpu.sync_copy(ib_hbm.at[pl.ds(off, sub)], ib_v.at[0])
            cpa = pltpu.async_copy(ta_hbm.at[ia_v.at[0]], ra_v, sema)
            cpb = pltpu.async_copy(tb_hbm.at[ib_v.at[0]], rb_v, semb)
            cpa.wait()
            cpb.wait()
            pltpu.sync_copy(ra_v, oa_hbm.at[pl.ds(off, sub)])
            pltpu.sync_copy(rb_v, ob_hbm.at[pl.ds(off, sub)])
            return carry

        lax.fori_loop(0, n_sub, body, 0)

    return gather_k(ta, tb, ia, ib)


def _bn_scale_shift(sumv, ssqv, count, g, b):
    mean = (sumv / count).reshape(-1)
    var = (ssqv / count).reshape(-1) - mean * mean
    inv = lax.rsqrt(var + _EPS)
    scale = g * inv
    shift = b - mean * scale
    return scale, shift


def _edge_gated_conv(p, x, edge_index, edge_attr, num_nodes,
                     blk_node, blk_edge):
    row = edge_index[0].astype(jnp.int32)
    col = edge_index[1].astype(jnp.int32)
    ne = edge_attr.shape[0]

    bg = p["src_gate"]["b"] + p["dst_gate"]["b"] + p["edge_gate"]["b"]
    gs, gd, ms = _precompute(x, p["src_gate"]["W"], p["dst_gate"]["W"],
                             p["src_msg"]["W"], p["src_msg"]["b"], blk_node)

    gsr, gdc = _sc_gather_pair(gs, gd, row, col, sub=200)

    h, hsum, hssq = _gate_pass(edge_attr, gsr, gdc, p["edge_gate"]["W"], bg,
                               blk_edge)
    e_scale, e_shift = _bn_scale_shift(hsum, hssq, float(ne),
                                       p["bn_edge"]["g"], p["bn_edge"]["b"])

    e_out, msg_e = _msg_pass(h, edge_attr, e_scale, e_shift,
                             p["edge_msg"]["W"], p["edge_msg"]["b"], blk_edge)

    msg = jnp.take(ms, row, axis=0) + msg_e
    agg = jax.ops.segment_sum(msg, col, num_segments=num_nodes)

    s, ssum, sssq = _silu_stats(agg, blk_node)
    n_scale, n_shift = _bn_scale_shift(ssum, sssq, float(num_nodes),
                                       p["bn_node"]["g"], p["bn_node"]["b"])
    x_out = _norm_res(s, x, n_scale, n_shift, blk_node)
    return x_out, e_out


def kernel(x, edge_index, edge_attr, line_index, line_attr, params):
    ea, la = _edge_gated_conv(params["line_conv"], edge_attr, line_index,
                              line_attr, edge_attr.shape[0],
                              blk_node=640, blk_edge=640)
    xo, ea = _edge_gated_conv(params["atom_conv"], x, edge_index, ea,
                              x.shape[0], blk_node=1000, blk_edge=640)
    return (xo, ea, la)


# speedup vs baseline: 1.2955x; 1.0339x over previous
import functools

import jax
import jax.numpy as jnp
from jax import lax
from jax.experimental import pallas as pl
from jax.experimental.pallas import tpu as pltpu
from jax.experimental.pallas import tpu_sc as plsc

_EPS = 1e-5

_NC = 2
_NS = 16
_NW = _NC * _NS


def _dotT(a, w):
    return lax.dot_general(a, w, (((1,), (1,)), ((), ())),
                           preferred_element_type=jnp.float32)


def _precompute_body(x_ref, wsg_ref, wdg_ref, wsm_ref, bsm_ref,
                     gs_ref, gd_ref, ms_ref):
    xb = x_ref[...]
    gs_ref[...] = _dotT(xb, wsg_ref[...])
    gd_ref[...] = _dotT(xb, wdg_ref[...])
    ms_ref[...] = _dotT(xb, wsm_ref[...]) + bsm_ref[...]


def _precompute(x, wsg, wdg, wsm, bsm, blk):
    n, din = x.shape
    de = wsg.shape[0]
    dn = wsm.shape[0]
    grid = (n // blk,)
    return pl.pallas_call(
        _precompute_body,
        grid=grid,
        in_specs=[
            pl.BlockSpec((blk, din), lambda i: (i, 0)),
            pl.BlockSpec((de, din), lambda i: (0, 0)),
            pl.BlockSpec((de, din), lambda i: (0, 0)),
            pl.BlockSpec((dn, din), lambda i: (0, 0)),
            pl.BlockSpec((1, dn), lambda i: (0, 0)),
        ],
        out_specs=[
            pl.BlockSpec((blk, de), lambda i: (i, 0)),
            pl.BlockSpec((blk, de), lambda i: (i, 0)),
            pl.BlockSpec((blk, dn), lambda i: (i, 0)),
        ],
        out_shape=[
            jax.ShapeDtypeStruct((n, de), jnp.float32),
            jax.ShapeDtypeStruct((n, de), jnp.float32),
            jax.ShapeDtypeStruct((n, dn), jnp.float32),
        ],
    )(x, wsg, wdg, wsm, bsm.reshape(1, -1))


def _gate_body(ea_ref, gsr_ref, gdc_ref, weg_ref, bg_ref,
               h_ref, sum_ref, ssq_ref):
    i = pl.program_id(0)
    ea = ea_ref[...]
    z = gsr_ref[...] + gdc_ref[...] + _dotT(ea, weg_ref[...]) + bg_ref[...]
    u = jax.nn.sigmoid(z) * ea
    h = u * jax.nn.sigmoid(u)
    h_ref[...] = h

    @pl.when(i == 0)
    def _():
        sum_ref[...] = jnp.zeros_like(sum_ref)
        ssq_ref[...] = jnp.zeros_like(ssq_ref)

    sum_ref[...] += jnp.sum(h, axis=0, keepdims=True)
    ssq_ref[...] += jnp.sum(h * h, axis=0, keepdims=True)


def _gate_pass(ea, gsr, gdc, weg, bg, blk):
    ne, de = ea.shape
    grid = (ne // blk,)
    return pl.pallas_call(
        _gate_body,
        grid=grid,
        in_specs=[
            pl.BlockSpec((blk, de), lambda i: (i, 0)),
            pl.BlockSpec((blk, de), lambda i: (i, 0)),
            pl.BlockSpec((blk, de), lambda i: (i, 0)),
            pl.BlockSpec((de, de), lambda i: (0, 0)),
            pl.BlockSpec((1, de), lambda i: (0, 0)),
        ],
        out_specs=[
            pl.BlockSpec((blk, de), lambda i: (i, 0)),
            pl.BlockSpec((1, de), lambda i: (0, 0)),
            pl.BlockSpec((1, de), lambda i: (0, 0)),
        ],
        out_shape=[
            jax.ShapeDtypeStruct((ne, de), jnp.float32),
            jax.ShapeDtypeStruct((1, de), jnp.float32),
            jax.ShapeDtypeStruct((1, de), jnp.float32),
        ],
    )(ea, gsr, gdc, weg, bg.reshape(1, -1))


def _msg_body(h_ref, ea_ref, scale_ref, shift_ref, wem_ref, bem_ref,
              eout_ref, msge_ref):
    en = h_ref[...] * scale_ref[...] + shift_ref[...]
    eout_ref[...] = ea_ref[...] + en
    msge_ref[...] = _dotT(en, wem_ref[...]) + bem_ref[...]


def _msg_pass(h, ea, scale, shift, wem, bem, blk):
    ne, de = h.shape
    dn = wem.shape[0]
    grid = (ne // blk,)
    return pl.pallas_call(
        _msg_body,
        grid=grid,
        in_specs=[
            pl.BlockSpec((blk, de), lambda i: (i, 0)),
            pl.BlockSpec((blk, de), lambda i: (i, 0)),
            pl.BlockSpec((1, de), lambda i: (0, 0)),
            pl.BlockSpec((1, de), lambda i: (0, 0)),
            pl.BlockSpec((dn, de), lambda i: (0, 0)),
            pl.BlockSpec((1, dn), lambda i: (0, 0)),
        ],
        out_specs=[
            pl.BlockSpec((blk, de), lambda i: (i, 0)),
            pl.BlockSpec((blk, dn), lambda i: (i, 0)),
        ],
        out_shape=[
            jax.ShapeDtypeStruct((ne, de), jnp.float32),
            jax.ShapeDtypeStruct((ne, dn), jnp.float32),
        ],
    )(h, ea, scale.reshape(1, -1), shift.reshape(1, -1), wem,
      bem.reshape(1, -1))


def _silu_stats_body(lo_ref, hi_ref, s_ref, sum_ref, ssq_ref):
    i = pl.program_id(0)
    a = jnp.concatenate([lo_ref[0], hi_ref[0]], axis=1)
    s = a * jax.nn.sigmoid(a)
    s_ref[...] = s

    @pl.when(i == 0)
    def _():
        sum_ref[...] = jnp.zeros_like(sum_ref)
        ssq_ref[...] = jnp.zeros_like(ssq_ref)

    sum_ref[...] += jnp.sum(s, axis=0, keepdims=True)
    ssq_ref[...] += jnp.sum(s * s, axis=0, keepdims=True)


def _silu_stats(agg2, blk):
    _, n, d2 = agg2.shape
    d = 2 * d2
    grid = (n // blk,)
    return pl.pallas_call(
        _silu_stats_body,
        grid=grid,
        in_specs=[
            pl.BlockSpec((1, blk, d2), lambda i: (0, i, 0)),
            pl.BlockSpec((1, blk, d2), lambda i: (1, i, 0)),
        ],
        out_specs=[
            pl.BlockSpec((blk, d), lambda i: (i, 0)),
            pl.BlockSpec((1, d), lambda i: (0, 0)),
            pl.BlockSpec((1, d), lambda i: (0, 0)),
        ],
        out_shape=[
            jax.ShapeDtypeStruct((n, d), jnp.float32),
            jax.ShapeDtypeStruct((1, d), jnp.float32),
            jax.ShapeDtypeStruct((1, d), jnp.float32),
        ],
    )(agg2, agg2)


def _norm_res_body(s_ref, x_ref, scale_ref, shift_ref, out_ref):
    out_ref[...] = x_ref[...] + s_ref[...] * scale_ref[...] + shift_ref[...]


def _norm_res(s, x, scale, shift, blk):
    n, d = s.shape
    grid = (n // blk,)
    return pl.pallas_call(
        _norm_res_body,
        grid=grid,
        in_specs=[
            pl.BlockSpec((blk, d), lambda i: (i, 0)),
            pl.BlockSpec((blk, d), lambda i: (i, 0)),
            pl.BlockSpec((1, d), lambda i: (0, 0)),
            pl.BlockSpec((1, d), lambda i: (0, 0)),
        ],
        out_specs=pl.BlockSpec((blk, d), lambda i: (i, 0)),
        out_shape=jax.ShapeDtypeStruct((n, d), jnp.float32),
    )(s, x, scale.reshape(1, -1), shift.reshape(1, -1))


def _sc_gather_pair(ta, tb, ia, ib, sub):
    b_total = ia.shape[0]
    da = ta.shape[1]
    db = tb.shape[1]
    b_per_w = b_total // _NW
    n_sub = b_per_w // sub
    mesh = plsc.VectorSubcoreMesh(core_axis_name="c", subcore_axis_name="s", num_cores=_NC, num_subcores=_NS)

    @functools.partial(
        pl.kernel,
        out_type=[
            jax.ShapeDtypeStruct((b_total, da), jnp.float32),
            jax.ShapeDtypeStruct((b_total, db), jnp.float32),
        ],
        mesh=mesh,
        scratch_types=[
            pltpu.VMEM((1, sub), jnp.int32),
            pltpu.VMEM((1, sub), jnp.int32),
            pltpu.VMEM((sub, da), jnp.float32),
            pltpu.VMEM((sub, db), jnp.float32),
            pltpu.SemaphoreType.DMA,
            pltpu.SemaphoreType.DMA,
        ],
    )
    def gather_k(ta_hbm, tb_hbm, ia_hbm, ib_hbm, oa_hbm, ob_hbm,
                 ia_v, ib_v, ra_v, rb_v, sema, semb):
        wid = lax.axis_index("s") * _NC + lax.axis_index("c")
        base = wid * b_per_w

        def body(j, carry):
            off = pl.multiple_of(base + j * sub, 8)
            pltpu.sync_copy(ia_hbm.at[pl.ds(off, sub)], ia_v.at[0])
            pltpu.sync_copy(ib_hbm.at[pl.ds(off, sub)], ib_v.at[0])
            cpa = pltpu.async_copy(ta_hbm.at[ia_v.at[0]], ra_v, sema)
            cpb = pltpu.async_copy(tb_hbm.at[ib_v.at[0]], rb_v, semb)
            cpa.wait()
            cpb.wait()
            pltpu.sync_copy(ra_v, oa_hbm.at[pl.ds(off, sub)])
            pltpu.sync_copy(rb_v, ob_hbm.at[pl.ds(off, sub)])
            return carry

        lax.fori_loop(0, n_sub, body, 0)

    return gather_k(ta, tb, ia, ib)


def _sc_segment_sum(ms2, msge2, row, col, nn, r_full, sub):
    ne = row.shape[0]
    d2 = 128
    zb = 16
    ept = ne // _NS
    n_full = nn // r_full
    r_last = nn - n_full * r_full
    n_chunks = n_full + (1 if r_last else 0)
    nsteps = ept // 16
    sub_shift = sub.bit_length() - 1
    base_rows = -(-r_full // 128) * 128
    nsub_max = (ept + sub - 1) // sub
    assert 1 << sub_shift == sub and ept % 16 == 0
    assert r_full % 128 == 0 and r_last % 8 == 0 and nn % 8 == 0
    mesh = plsc.VectorSubcoreMesh(core_axis_name="c", subcore_axis_name="s", num_cores=_NC, num_subcores=_NS)

    def _wb_sizes(rc):
        main = -(-(-(-rc // _NS) // 8) * 8)
        main = ((rc // _NS + 7) // 8) * 8
        tail = rc - (_NS - 1) * main
        assert tail % 8 == 0 and 0 < tail <= main
        return main, tail

    @functools.partial(
        pl.kernel,
        out_type=jax.ShapeDtypeStruct((2, nn, d2), jnp.float32),
        mesh=mesh,
        compiler_params=pltpu.CompilerParams(needs_layout_passes=False),
        scratch_types=[
            pltpu.VMEM((ept,), jnp.int32),
            pltpu.VMEM((ept,), jnp.int32),
            pltpu.VMEM((ept + 2 * sub,), jnp.int32),
            pltpu.VMEM((sub,), jnp.int32),
            pltpu.VMEM((sub,), jnp.int32),
            pltpu.VMEM((sub,), jnp.int32),
            pltpu.VMEM((sub,), jnp.int32),
            pltpu.VMEM((sub,), jnp.int32),
            pltpu.VMEM((sub,), jnp.int32),
            pltpu.VMEM((sub, d2), jnp.float32),
            pltpu.VMEM((sub, d2), jnp.float32),
            pltpu.VMEM((sub, d2), jnp.float32),
            pltpu.VMEM((sub, d2), jnp.float32),
            pltpu.VMEM((zb, d2), jnp.float32),
            pltpu.VMEM_SHARED((base_rows + _NS, d2), jnp.float32),
            pltpu.SemaphoreType.DMA,
            pltpu.SemaphoreType.DMA,
            pltpu.SemaphoreType.DMA,
            pltpu.SemaphoreType.DMA,
            pltpu.SemaphoreType.DMA,
        ],
    )
    def scat_k(ms_hbm, me_hbm, row_hbm, col_hbm, z_hbm, out_hbm,
               col_v, row_v, ebuf, r0, e0, d0, r1, e1, d1,
               buf_a0, buf_b0, buf_a1, buf_b1, zbuf, acc,
               sem_g0, sem_g1, sem_s0, sem_s1, sem_z):
        cid = lax.axis_index("c")
        sid = lax.axis_index("s")
        e_base = sid * ept
        iota = lax.iota(jnp.int32, 16)
        sets = ((r0, e0, d0, buf_a0, buf_b0, sem_g0, sem_s0),
                (r1, e1, d1, buf_a1, buf_b1, sem_g1, sem_s1))

        pltpu.sync_copy(col_hbm.at[pl.ds(e_base, ept)], col_v)
        pltpu.sync_copy(row_hbm.at[pl.ds(e_base, ept)], row_v)
        pltpu.sync_copy(z_hbm, zbuf)

        def chunk_body(k, carry):
            lo = k * r_full
            rc = jnp.where(k < n_full, r_full, r_last)
            hi = lo + rc

            nblk = lax.shift_right_logical(rc + (zb - 1), 4)
            nt = -(-base_rows // zb // _NS) + 1
            for t in range(nt):
                blk = sid + t * _NS

                @pl.when(blk < nblk)
                def _():
                    pltpu.sync_copy(zbuf, acc.at[pl.ds(blk * zb, zb)])

            unroll = 5
            assert nsteps % unroll == 0

            def scan(g, ptr_vec):
                for u in range(unroll):
                    stp = g * unroll + u
                    cv = col_v[pl.ds(stp * 16, 16)]
                    mk = (cv >= lo) & (cv < hi)
                    csum = plsc.cumsum(mk.astype(jnp.int32))
                    epos = e_base + stp * 16 + iota
                    plsc.store_scatter(ebuf, [ptr_vec + csum - 1], epos,
                                       mask=mk)
                    ptr_vec = ptr_vec + plsc.all_reduce_population_count(mk)
                return ptr_vec
            ptr_vec = lax.fori_loop(0, nsteps // unroll, scan,
                                    jnp.zeros((16,), jnp.int32))
            m = ptr_vec[0]

            plsc.subcore_barrier()

            nsub = lax.shift_right_logical(m + (sub - 1), sub_shift)

            def build(st, j):
                rs, es, ds, _a, _b, _g, _s = sets[st]
                jbase = j * sub
                for q in range(sub // 16):
                    pos = jbase + q * 16 + iota
                    ev = ebuf[pl.ds(jbase + q * 16, 16)]
                    valid = pos < m
                    el = ev - e_base
                    cg = plsc.load_gather(col_v, [el], mask=valid)
                    rg = plsc.load_gather(row_v, [el], mask=valid)
                    pad = (sid * 16 + iota) * 2 + cid
                    ds[pl.ds(q * 16, 16)] = jnp.where(
                        valid, cg - lo, base_rows + sid)
                    rs[pl.ds(q * 16, 16)] = jnp.where(
                        valid, 2 * rg + cid, pad)
                    es[pl.ds(q * 16, 16)] = jnp.where(
                        valid, 2 * ev + cid, pad)

            def issue_g(st):
                rs, es, ds, ba, bb, sg, _s = sets[st]
                pltpu.async_copy(ms_hbm.at[rs], ba, sg)
                pltpu.async_copy(me_hbm.at[es], bb, sg)

            def drain_g(st):
                rs, es, ds, ba, bb, sg, _s = sets[st]
                pltpu.make_async_copy(ms_hbm.at[rs], ba, sg).wait()
                pltpu.make_async_copy(me_hbm.at[es], bb, sg).wait()

            def issue_s(st):
                rs, es, ds, ba, bb, _g, ss = sets[st]
                pltpu.async_copy(ba, acc.at[ds], ss, add=True)
                pltpu.async_copy(bb, acc.at[ds], ss, add=True)

            def drain_s(st):
                rs, es, ds, ba, bb, _g, ss = sets[st]
                pltpu.make_async_copy(ba, acc.at[ds], ss).wait()
                pltpu.make_async_copy(bb, acc.at[ds], ss).wait()

            def dma(j, c2):
                build(0, j)
                issue_g(0)
                drain_g(0)
                rs, es, ds, ba, bb, _g, _ss = sets[0]
                pltpu.sync_copy(ba, acc.at[ds], add=True)
                pltpu.sync_copy(bb, acc.at[ds], add=True)
                return c2
            lax.fori_loop(0, nsub, dma, 0)

            plsc.subcore_barrier()

            def wb(rc_static):
                main, tail = _wb_sizes(rc_static)

                @pl.when(sid < _NS - 1)
                def _():
                    pltpu.sync_copy(
                        acc.at[pl.ds(sid * main, main)],
                        out_hbm.at[cid, pl.ds(lo + sid * main, main)])

                @pl.when(sid == _NS - 1)
                def _():
                    pltpu.sync_copy(
                        acc.at[pl.ds((_NS - 1) * main, tail)],
                        out_hbm.at[cid, pl.ds(lo + (_NS - 1) * main, tail)])

            @pl.when(k < n_full)
            def _():
                wb(r_full)

            if r_last:
                @pl.when(k == n_full)
                def _():
                    wb(r_last)

            plsc.subcore_barrier()
            return carry

        lax.fori_loop(0, n_chunks, chunk_body, 0)

    zeros = jnp.zeros((zb, d2), jnp.float32)
    return scat_k(ms2, msge2, row, col, zeros)


def _bn_scale_shift(sumv, ssqv, count, g, b):
    mean = (sumv / count).reshape(-1)
    var = (ssqv / count).reshape(-1) - mean * mean
    inv = lax.rsqrt(var + _EPS)
    scale = g * inv
    shift = b - mean * scale
    return scale, shift


def _edge_gated_conv(p, x, edge_index, edge_attr, num_nodes,
                     blk_node, blk_edge, r_full):
    row = edge_index[0].astype(jnp.int32)
    col = edge_index[1].astype(jnp.int32)
    ne = edge_attr.shape[0]

    bg = p["src_gate"]["b"] + p["dst_gate"]["b"] + p["edge_gate"]["b"]
    gs, gd, ms = _precompute(x, p["src_gate"]["W"], p["dst_gate"]["W"],
                             p["src_msg"]["W"], p["src_msg"]["b"], blk_node)

    gsr, gdc = _sc_gather_pair(gs, gd, row, col, sub=200)

    h, hsum, hssq = _gate_pass(edge_attr, gsr, gdc, p["edge_gate"]["W"], bg,
                               blk_edge)
    e_scale, e_shift = _bn_scale_shift(hsum, hssq, float(ne),
                                       p["bn_edge"]["g"], p["bn_edge"]["b"])

    e_out, msg_e = _msg_pass(h, edge_attr, e_scale, e_shift,
                             p["edge_msg"]["W"], p["edge_msg"]["b"], blk_edge)

    agg2 = _sc_segment_sum(ms.reshape(2 * num_nodes, 128),
                           msg_e.reshape(2 * ne, 128),
                           row, col, num_nodes, r_full, sub=32)

    s, ssum, sssq = _silu_stats(agg2, blk_node)
    n_scale, n_shift = _bn_scale_shift(ssum, sssq, float(num_nodes),
                                       p["bn_node"]["g"], p["bn_node"]["b"])
    x_out = _norm_res(s, x, n_scale, n_shift, blk_node)
    return x_out, e_out


def kernel(x, edge_index, edge_attr, line_index, line_attr, params):
    ea, la = _edge_gated_conv(params["line_conv"], edge_attr, line_index,
                              line_attr, edge_attr.shape[0],
                              blk_node=640, blk_edge=640,
                              r_full=6400)
    xo, ea = _edge_gated_conv(params["atom_conv"], x, edge_index, ea,
                              x.shape[0], blk_node=1000, blk_edge=640,
                              r_full=6400)
    return (xo, ea, la)
